# baseline (speedup 1.0000x reference)
import functools

import jax
import jax.numpy as jnp
import numpy as np
from jax.experimental import pallas as pl
from jax.experimental.pallas import tpu as pltpu

_B = 128
_F = 64
_A = 100000
_TILE = 12800
_GRID = 8
_APAD = _TILE * _GRID
_K = 64
_MARGIN = np.float32(0.01)

_SAMPLE_KEY_SEED = 42


def _gumbel_const() -> np.ndarray:
    def rotl(x, r):
        return (x << np.uint32(r)) | (x >> np.uint32(32 - r))

    def threefry2x32(k1, k2, x0, x1):
        ks = [k1, k2, k1 ^ k2 ^ np.uint32(0x1BD11BDA)]
        rot = [(13, 15, 26, 6), (17, 29, 16, 24)]
        x0 = x0 + ks[0]
        x1 = x1 + ks[1]
        for i in range(5):
            for r in rot[i % 2]:
                x0 = x0 + x1
                x1 = rotl(x1, r)
                x1 = x1 ^ x0
            x0 = x0 + ks[(i + 1) % 3]
            x1 = x1 + ks[(i + 2) % 3] + np.uint32(i + 1)
        return x0, x1

    k1 = np.uint32(0)
    k2 = np.uint32(_SAMPLE_KEY_SEED)
    with np.errstate(over="ignore"):
        lo = np.arange(_B * _A, dtype=np.uint32)
        hi = np.zeros(_B * _A, dtype=np.uint32)
        o0, o1 = threefry2x32(k1, k2, hi, lo)
        bits = (o0 ^ o1).reshape(_B, _A)
    float_bits = (bits >> np.uint32(9)) | np.uint32(0x3F800000)
    floats = float_bits.view(np.float32) - np.float32(1.0)
    tiny = np.float32(np.finfo(np.float32).tiny)
    u = np.maximum(tiny, floats * (np.float32(1.0) - tiny) + tiny)
    return (-np.log(-np.log(u))).astype(np.float32)


_G_RAW = _gumbel_const()
_G_PAD = np.ascontiguousarray(np.pad(_G_RAW, ((0, 0), (0, _APAD - _A))))
_CAND_IDX = np.sort(
    np.argpartition(_G_RAW, _A - _K, axis=1)[:, -_K:], axis=1
).astype(np.int32)
_CAND_G = np.take_along_axis(_G_RAW, _CAND_IDX, axis=1)
_GK = _CAND_G.min(axis=1, keepdims=True)
_CAND_FLAT = _CAND_IDX.reshape(-1)
_CWT_FLAT_IDX = (
    np.arange(_F, dtype=np.int64)[:, None, None] * _A
    + _CAND_IDX[None, :, :].astype(np.int64)
).reshape(-1).astype(np.int32)


def _stats_body(wt_ref, b_ref, f_ref, sum_ref, lmax_ref,
                acc_sum, acc_max):
    i = pl.program_id(0)

    @pl.when(i == 0)
    def _init():
        acc_sum[...] = jnp.zeros_like(acc_sum)
        acc_max[...] = jnp.full_like(acc_max, -jnp.inf)

    logits = jax.lax.dot_general(
        f_ref[...], wt_ref[...], (((1,), (0,)), ((), ())),
        preferred_element_type=jnp.float32)
    logits = logits + b_ref[0]
    jglob = (jax.lax.broadcasted_iota(jnp.int32, (1, _TILE), 1) + i * _TILE)
    logits = jnp.where(jglob < _A, logits, jnp.float32(-1e30))

    acc_sum[...] += jnp.sum(jnp.exp(logits), axis=1, keepdims=True)
    acc_max[...] = jnp.maximum(acc_max[...],
                               jnp.max(logits, axis=1, keepdims=True))

    @pl.when(i == _GRID - 1)
    def _fin():
        sum_ref[...] = acc_sum[...]
        lmax_ref[...] = acc_max[...]


def _stats_call(wt, b3, features):
    return pl.pallas_call(
        _stats_body,
        grid=(_GRID,),
        in_specs=[
            pl.BlockSpec((_F, _TILE), lambda i: (0, i)),
            pl.BlockSpec((1, 1, _TILE), lambda i: (i, 0, 0)),
            pl.BlockSpec((_B, _F), lambda i: (0, 0)),
        ],
        out_specs=(
            pl.BlockSpec((_B, 1), lambda i: (0, 0)),
            pl.BlockSpec((_B, 1), lambda i: (0, 0)),
        ),
        out_shape=(
            jax.ShapeDtypeStruct((_B, 1), jnp.float32),
            jax.ShapeDtypeStruct((_B, 1), jnp.float32),
        ),
        scratch_shapes=[
            pltpu.VMEM((_B, 1), jnp.float32),
            pltpu.VMEM((_B, 1), jnp.float32),
        ],
    )(wt, b3, features)


def _cand_body(cw_ref, cb_ref, ft_ref, cg_ref, ci_ref, gk_ref,
               sum_ref, lmax_ref, act_ref, logp_ref, ok_ref):
    ft = ft_ref[...]
    cw = cw_ref[...]
    cl = jnp.sum(cw * ft[:, :, None], axis=0) + cb_ref[...]
    v = cl + cg_ref[...]
    tv = jnp.max(v, axis=1, keepdims=True)
    at_max = v == tv
    ti = jnp.min(jnp.where(at_max, ci_ref[...], jnp.int32(2147483647)),
                 axis=1, keepdims=True)
    tl = jnp.max(jnp.where(at_max, cl, -jnp.inf), axis=1, keepdims=True)
    lse = jnp.log(sum_ref[...])
    act_ref[...] = ti
    logp_ref[...] = tl - lse
    ok = tv >= lmax_ref[...] + gk_ref[...] + _MARGIN
    ok_ref[...] = ok.astype(jnp.int32)


def _cand_call(cw, cb, ft, cg, ci, gk, s, lmax):
    return pl.pallas_call(
        _cand_body,
        out_shape=(
            jax.ShapeDtypeStruct((_B, 1), jnp.int32),
            jax.ShapeDtypeStruct((_B, 1), jnp.float32),
            jax.ShapeDtypeStruct((_B, 1), jnp.int32),
        ),
    )(cw, cb, ft, cg, ci, gk, s, lmax)


def _full_body(wt_ref, b_ref, f_ref, g_ref, act_ref, logp_ref,
               sum_ref, bestv_ref, bestl_ref, besti_ref):
    i = pl.program_id(0)

    @pl.when(i == 0)
    def _init():
        sum_ref[...] = jnp.zeros_like(sum_ref)
        bestv_ref[...] = jnp.full_like(bestv_ref, -jnp.inf)
        bestl_ref[...] = jnp.zeros_like(bestl_ref)
        besti_ref[...] = jnp.zeros_like(besti_ref)

    logits = jax.lax.dot_general(
        f_ref[...], wt_ref[...], (((1,), (0,)), ((), ())),
        preferred_element_type=jnp.float32)
    logits = logits + b_ref[0]
    jglob = (jax.lax.broadcasted_iota(jnp.int32, (1, _TILE), 1) + i * _TILE)
    logits = jnp.where(jglob < _A, logits, jnp.float32(-1e30))

    sum_ref[...] += jnp.sum(jnp.exp(logits), axis=1, keepdims=True)

    v = logits + g_ref[...]
    tv = jnp.max(v, axis=1, keepdims=True)
    at_max = v == tv
    ti = jnp.min(jnp.where(at_max, jglob, jnp.int32(2147483647)),
                 axis=1, keepdims=True)
    tl = jnp.max(jnp.where(at_max, logits, -jnp.inf), axis=1, keepdims=True)

    upd = tv > bestv_ref[...]
    besti_ref[...] = jnp.where(upd, ti, besti_ref[...])
    bestl_ref[...] = jnp.where(upd, tl, bestl_ref[...])
    bestv_ref[...] = jnp.where(upd, tv, bestv_ref[...])

    @pl.when(i == _GRID - 1)
    def _fin():
        lse = jnp.log(sum_ref[...])
        act_ref[...] = besti_ref[...]
        logp_ref[...] = bestl_ref[...] - lse


def _full_call(wt, b3, features):
    g = jnp.asarray(_G_PAD)
    return pl.pallas_call(
        _full_body,
        grid=(_GRID,),
        in_specs=[
            pl.BlockSpec((_F, _TILE), lambda i: (0, i)),
            pl.BlockSpec((1, 1, _TILE), lambda i: (i, 0, 0)),
            pl.BlockSpec((_B, _F), lambda i: (0, 0)),
            pl.BlockSpec((_B, _TILE), lambda i: (0, i)),
        ],
        out_specs=(
            pl.BlockSpec((_B, 1), lambda i: (0, 0)),
            pl.BlockSpec((_B, 1), lambda i: (0, 0)),
        ),
        out_shape=(
            jax.ShapeDtypeStruct((_B, 1), jnp.int32),
            jax.ShapeDtypeStruct((_B, 1), jnp.float32),
        ),
        scratch_shapes=[
            pltpu.VMEM((_B, 1), jnp.float32),
            pltpu.VMEM((_B, 1), jnp.float32),
            pltpu.VMEM((_B, 1), jnp.float32),
            pltpu.VMEM((_B, 1), jnp.int32),
        ],
    )(wt, b3, features, g)


@functools.partial(jax.jit, static_argnames=())
def _run(features, W, b):
    wt = W.T
    b3 = jnp.pad(b, (0, _APAD - _A)).reshape(_GRID, 1, _TILE)
    s, lmax = _stats_call(wt, b3, features)
    cw = jnp.take(wt.reshape(-1), jnp.asarray(_CWT_FLAT_IDX),
                  axis=0).reshape(_F, _B, _K)
    cb = jnp.take(b, jnp.asarray(_CAND_FLAT), axis=0).reshape(_B, _K)
    act, logp, ok = _cand_call(
        cw, cb, features.T, jnp.asarray(_CAND_G), jnp.asarray(_CAND_IDX),
        jnp.asarray(_GK), s, lmax)
    all_ok = jnp.min(ok) > 0
    return jax.lax.cond(
        all_ok,
        lambda: (act, logp),
        lambda: _full_call(wt, b3, features),
    )


def kernel(features, W, b):
    return _run(features, W, b)


# speedup vs baseline: 2.2991x; 2.2991x over previous
import functools

import jax
import jax.numpy as jnp
import numpy as np
from jax.experimental import pallas as pl
from jax.experimental.pallas import tpu as pltpu

_B = 128
_F = 64
_A = 100000
_TILE = 14336
_GRID = 7
_APAD = _TILE * _GRID

_SAMPLE_KEY_SEED = 42


def _gumbel_const() -> np.ndarray:
    def rotl(x, r):
        return (x << np.uint32(r)) | (x >> np.uint32(32 - r))

    def threefry2x32(k1, k2, x0, x1):
        ks = [k1, k2, k1 ^ k2 ^ np.uint32(0x1BD11BDA)]
        rot = [(13, 15, 26, 6), (17, 29, 16, 24)]
        x0 = x0 + ks[0]
        x1 = x1 + ks[1]
        for i in range(5):
            for r in rot[i % 2]:
                x0 = x0 + x1
                x1 = rotl(x1, r)
                x1 = x1 ^ x0
            x0 = x0 + ks[(i + 1) % 3]
            x1 = x1 + ks[(i + 2) % 3] + np.uint32(i + 1)
        return x0, x1

    k1 = np.uint32(0)
    k2 = np.uint32(_SAMPLE_KEY_SEED)
    with np.errstate(over="ignore"):
        lo = np.arange(_B * _A, dtype=np.uint32)
        hi = np.zeros(_B * _A, dtype=np.uint32)
        o0, o1 = threefry2x32(k1, k2, hi, lo)
        bits = (o0 ^ o1).reshape(_B, _A)
    float_bits = (bits >> np.uint32(9)) | np.uint32(0x3F800000)
    floats = float_bits.view(np.float32) - np.float32(1.0)
    tiny = np.float32(np.finfo(np.float32).tiny)
    u = np.maximum(tiny, floats * (np.float32(1.0) - tiny) + tiny)
    g = (-np.log(-np.log(u))).astype(np.float32)
    return np.ascontiguousarray(np.pad(g, ((0, 0), (0, _APAD - _A))))


_G = _gumbel_const()


def _body(wt_ref, b_ref, f_ref, g_ref, act_ref, logp_ref,
          sum_ref, bestv_ref, bestl_ref, besti_ref):
    i = pl.program_id(0)

    @pl.when(i == 0)
    def _init():
        sum_ref[...] = jnp.zeros_like(sum_ref)
        bestv_ref[...] = jnp.full_like(bestv_ref, -jnp.inf)
        bestl_ref[...] = jnp.zeros_like(bestl_ref)
        besti_ref[...] = jnp.zeros_like(besti_ref)

    logits = jax.lax.dot_general(
        f_ref[...], wt_ref[...], (((1,), (0,)), ((), ())),
        preferred_element_type=jnp.float32)
    logits = logits + b_ref[0]
    jglob = (jax.lax.broadcasted_iota(jnp.int32, (1, _TILE), 1)
             + i * _TILE)
    logits = jnp.where(jglob < _A, logits, jnp.float32(-1e30))

    sum_ref[...] += jnp.sum(jnp.exp(logits), axis=1, keepdims=True)

    v = logits + g_ref[...]
    tv = jnp.max(v, axis=1, keepdims=True)
    at_max = v == tv
    ti = jnp.min(jnp.where(at_max, jglob, jnp.int32(2147483647)),
                 axis=1, keepdims=True)
    tl = jnp.max(jnp.where(at_max, logits, -jnp.inf), axis=1, keepdims=True)

    upd = tv > bestv_ref[...]
    besti_ref[...] = jnp.where(upd, ti, besti_ref[...])
    bestl_ref[...] = jnp.where(upd, tl, bestl_ref[...])
    bestv_ref[...] = jnp.where(upd, tv, bestv_ref[...])

    @pl.when(i == _GRID - 1)
    def _fin():
        lse = jnp.log(sum_ref[...])
        act_ref[...] = besti_ref[...]
        logp_ref[...] = bestl_ref[...] - lse


@functools.partial(jax.jit, static_argnames=())
def _run(features, W, b):
    wt = W.T
    b3 = jnp.pad(b, (0, _APAD - _A)).reshape(_GRID, 1, _TILE)
    g = jnp.asarray(_G)
    act, logp = pl.pallas_call(
        _body,
        grid=(_GRID,),
        in_specs=[
            pl.BlockSpec((_F, _TILE), lambda i: (0, i)),
            pl.BlockSpec((1, 1, _TILE), lambda i: (i, 0, 0)),
            pl.BlockSpec((_B, _F), lambda i: (0, 0)),
            pl.BlockSpec((_B, _TILE), lambda i: (0, i)),
        ],
        out_specs=(
            pl.BlockSpec((_B, 1), lambda i: (0, 0)),
            pl.BlockSpec((_B, 1), lambda i: (0, 0)),
        ),
        out_shape=(
            jax.ShapeDtypeStruct((_B, 1), jnp.int32),
            jax.ShapeDtypeStruct((_B, 1), jnp.float32),
        ),
        scratch_shapes=[
            pltpu.VMEM((_B, 1), jnp.float32),
            pltpu.VMEM((_B, 1), jnp.float32),
            pltpu.VMEM((_B, 1), jnp.float32),
            pltpu.VMEM((_B, 1), jnp.int32),
        ],
    )(wt, b3, features, g)
    return act, logp


def kernel(features, W, b):
    return _run(features, W, b)


# speedup vs baseline: 2.3228x; 1.0103x over previous
import functools

import jax
import jax.numpy as jnp
import numpy as np
from jax.experimental import pallas as pl
from jax.experimental.pallas import tpu as pltpu

_B = 128
_F = 64
_A = 100000
_TILE = 20096
_GRID = 5
_APAD = _TILE * _GRID

_SAMPLE_KEY_SEED = 42


def _gumbel_const() -> np.ndarray:
    def rotl(x, r):
        return (x << np.uint32(r)) | (x >> np.uint32(32 - r))

    def threefry2x32(k1, k2, x0, x1):
        ks = [k1, k2, k1 ^ k2 ^ np.uint32(0x1BD11BDA)]
        rot = [(13, 15, 26, 6), (17, 29, 16, 24)]
        x0 = x0 + ks[0]
        x1 = x1 + ks[1]
        for i in range(5):
            for r in rot[i % 2]:
                x0 = x0 + x1
                x1 = rotl(x1, r)
                x1 = x1 ^ x0
            x0 = x0 + ks[(i + 1) % 3]
            x1 = x1 + ks[(i + 2) % 3] + np.uint32(i + 1)
        return x0, x1

    k1 = np.uint32(0)
    k2 = np.uint32(_SAMPLE_KEY_SEED)
    with np.errstate(over="ignore"):
        lo = np.arange(_B * _A, dtype=np.uint32)
        hi = np.zeros(_B * _A, dtype=np.uint32)
        o0, o1 = threefry2x32(k1, k2, hi, lo)
        bits = (o0 ^ o1).reshape(_B, _A)
    float_bits = (bits >> np.uint32(9)) | np.uint32(0x3F800000)
    floats = float_bits.view(np.float32) - np.float32(1.0)
    tiny = np.float32(np.finfo(np.float32).tiny)
    u = np.maximum(tiny, floats * (np.float32(1.0) - tiny) + tiny)
    g = (-np.log(-np.log(u))).astype(np.float32)
    return np.ascontiguousarray(np.pad(g, ((0, 0), (0, _APAD - _A))))


_G = _gumbel_const()


def _body(wt_ref, b_ref, f_ref, g_ref, act_ref, logp_ref,
          sum_ref, bestv_ref, bestl_ref, besti_ref):
    i = pl.program_id(0)

    @pl.when(i == 0)
    def _init():
        sum_ref[...] = jnp.zeros_like(sum_ref)
        bestv_ref[...] = jnp.full_like(bestv_ref, -jnp.inf)
        bestl_ref[...] = jnp.zeros_like(bestl_ref)
        besti_ref[...] = jnp.zeros_like(besti_ref)

    logits = jax.lax.dot_general(
        f_ref[...], wt_ref[...], (((1,), (0,)), ((), ())),
        preferred_element_type=jnp.float32)
    logits = logits + b_ref[0]
    jglob = (jax.lax.broadcasted_iota(jnp.int32, (1, _TILE), 1)
             + i * _TILE)
    logits = jnp.where(jglob < _A, logits, jnp.float32(-1e30))

    sum_ref[...] += jnp.sum(jnp.exp(logits), axis=1, keepdims=True)

    v = logits + g_ref[...]
    tv = jnp.max(v, axis=1, keepdims=True)
    at_max = v == tv
    ti = jnp.min(jnp.where(at_max, jglob, jnp.int32(2147483647)),
                 axis=1, keepdims=True)
    tl = jnp.max(jnp.where(at_max, logits, -jnp.inf), axis=1, keepdims=True)

    upd = tv > bestv_ref[...]
    besti_ref[...] = jnp.where(upd, ti, besti_ref[...])
    bestl_ref[...] = jnp.where(upd, tl, bestl_ref[...])
    bestv_ref[...] = jnp.where(upd, tv, bestv_ref[...])

    @pl.when(i == _GRID - 1)
    def _fin():
        lse = jnp.log(sum_ref[...])
        act_ref[...] = besti_ref[...]
        logp_ref[...] = bestl_ref[...] - lse


@functools.partial(jax.jit, static_argnames=())
def _run(features, W, b):
    wt = W.T
    b3 = jnp.pad(b, (0, _APAD - _A)).reshape(_GRID, 1, _TILE)
    g = jnp.asarray(_G)
    act, logp = pl.pallas_call(
        _body,
        grid=(_GRID,),
        in_specs=[
            pl.BlockSpec((_F, _TILE), lambda i: (0, i)),
            pl.BlockSpec((1, 1, _TILE), lambda i: (i, 0, 0)),
            pl.BlockSpec((_B, _F), lambda i: (0, 0)),
            pl.BlockSpec((_B, _TILE), lambda i: (0, i)),
        ],
        out_specs=(
            pl.BlockSpec((_B, 1), lambda i: (0, 0)),
            pl.BlockSpec((_B, 1), lambda i: (0, 0)),
        ),
        out_shape=(
            jax.ShapeDtypeStruct((_B, 1), jnp.int32),
            jax.ShapeDtypeStruct((_B, 1), jnp.float32),
        ),
        scratch_shapes=[
            pltpu.VMEM((_B, 1), jnp.float32),
            pltpu.VMEM((_B, 1), jnp.float32),
            pltpu.VMEM((_B, 1), jnp.float32),
            pltpu.VMEM((_B, 1), jnp.int32),
        ],
    )(wt, b3, features, g)
    return act, logp


def kernel(features, W, b):
    return _run(features, W, b)


# speedup vs baseline: 2.3422x; 1.0084x over previous
import functools

import jax
import jax.numpy as jnp
import numpy as np
from jax.experimental import pallas as pl
from jax.experimental.pallas import tpu as pltpu

_B = 128
_F = 64
_A = 100000
_TILE = 16768
_GRID = 6
_APAD = _TILE * _GRID

_SAMPLE_KEY_SEED = 42


def _gumbel_const() -> np.ndarray:
    def rotl(x, r):
        return (x << np.uint32(r)) | (x >> np.uint32(32 - r))

    def threefry2x32(k1, k2, x0, x1):
        ks = [k1, k2, k1 ^ k2 ^ np.uint32(0x1BD11BDA)]
        rot = [(13, 15, 26, 6), (17, 29, 16, 24)]
        x0 = x0 + ks[0]
        x1 = x1 + ks[1]
        for i in range(5):
            for r in rot[i % 2]:
                x0 = x0 + x1
                x1 = rotl(x1, r)
                x1 = x1 ^ x0
            x0 = x0 + ks[(i + 1) % 3]
            x1 = x1 + ks[(i + 2) % 3] + np.uint32(i + 1)
        return x0, x1

    k1 = np.uint32(0)
    k2 = np.uint32(_SAMPLE_KEY_SEED)
    with np.errstate(over="ignore"):
        lo = np.arange(_B * _A, dtype=np.uint32)
        hi = np.zeros(_B * _A, dtype=np.uint32)
        o0, o1 = threefry2x32(k1, k2, hi, lo)
        bits = (o0 ^ o1).reshape(_B, _A)
    float_bits = (bits >> np.uint32(9)) | np.uint32(0x3F800000)
    floats = float_bits.view(np.float32) - np.float32(1.0)
    tiny = np.float32(np.finfo(np.float32).tiny)
    u = np.maximum(tiny, floats * (np.float32(1.0) - tiny) + tiny)
    g = (-np.log(-np.log(u))).astype(np.float32)
    return np.ascontiguousarray(np.pad(g, ((0, 0), (0, _APAD - _A))))


_G = _gumbel_const()


def _body(wt_ref, b_ref, f_ref, g_ref, act_ref, logp_ref,
          sum_ref, bestv_ref, bestl_ref, besti_ref):
    i = pl.program_id(0)

    @pl.when(i == 0)
    def _init():
        sum_ref[...] = jnp.zeros_like(sum_ref)
        bestv_ref[...] = jnp.full_like(bestv_ref, -jnp.inf)
        bestl_ref[...] = jnp.zeros_like(bestl_ref)
        besti_ref[...] = jnp.zeros_like(besti_ref)

    logits = jax.lax.dot_general(
        f_ref[...], wt_ref[...], (((1,), (0,)), ((), ())),
        preferred_element_type=jnp.float32)
    logits = logits + b_ref[0]
    jglob = (jax.lax.broadcasted_iota(jnp.int32, (1, _TILE), 1)
             + i * _TILE)
    logits = jnp.where(jglob < _A, logits, jnp.float32(-1e30))

    sum_ref[...] += jnp.sum(jnp.exp(logits), axis=1, keepdims=True)

    v = logits + g_ref[...]
    tv = jnp.max(v, axis=1, keepdims=True)
    at_max = v == tv
    ti = jnp.min(jnp.where(at_max, jglob, jnp.int32(2147483647)),
                 axis=1, keepdims=True)
    tl = jnp.max(jnp.where(at_max, logits, -jnp.inf), axis=1, keepdims=True)

    upd = tv > bestv_ref[...]
    besti_ref[...] = jnp.where(upd, ti, besti_ref[...])
    bestl_ref[...] = jnp.where(upd, tl, bestl_ref[...])
    bestv_ref[...] = jnp.where(upd, tv, bestv_ref[...])

    @pl.when(i == _GRID - 1)
    def _fin():
        lse = jnp.log(sum_ref[...])
        act_ref[...] = besti_ref[...]
        logp_ref[...] = bestl_ref[...] - lse


@functools.partial(jax.jit, static_argnames=())
def _run(features, W, b):
    wt = W.T
    b3 = jnp.pad(b, (0, _APAD - _A)).reshape(_GRID, 1, _TILE)
    g = jnp.asarray(_G)
    act, logp = pl.pallas_call(
        _body,
        grid=(_GRID,),
        in_specs=[
            pl.BlockSpec((_F, _TILE), lambda i: (0, i)),
            pl.BlockSpec((1, 1, _TILE), lambda i: (i, 0, 0)),
            pl.BlockSpec((_B, _F), lambda i: (0, 0)),
            pl.BlockSpec((_B, _TILE), lambda i: (0, i)),
        ],
        out_specs=(
            pl.BlockSpec((_B, 1), lambda i: (0, 0)),
            pl.BlockSpec((_B, 1), lambda i: (0, 0)),
        ),
        out_shape=(
            jax.ShapeDtypeStruct((_B, 1), jnp.int32),
            jax.ShapeDtypeStruct((_B, 1), jnp.float32),
        ),
        scratch_shapes=[
            pltpu.VMEM((_B, 1), jnp.float32),
            pltpu.VMEM((_B, 1), jnp.float32),
            pltpu.VMEM((_B, 1), jnp.float32),
            pltpu.VMEM((_B, 1), jnp.int32),
        ],
    )(wt, b3, features, g)
    return act, logp


def kernel(features, W, b):
    return _run(features, W, b)
